# baseline (speedup 1.0000x reference)
import functools

import jax
import jax.numpy as jnp
from jax import lax
from jax.experimental import pallas as pl
from jax.experimental.pallas import tpu as pltpu
from jax.experimental.pallas import tpu_sc as plsc

_CHUNK = 128


@functools.lru_cache(maxsize=None)
def _build(n_user, n_item, batch, embed):
    info = plsc.get_sparse_core_info()
    num_cores = info.num_cores
    num_workers = info.num_cores * info.num_subcores
    assert batch % (num_workers * _CHUNK) == 0
    b_per_w = batch // num_workers
    n_chunks = b_per_w // _CHUNK
    total = 2 * batch

    mesh = plsc.VectorSubcoreMesh(core_axis_name="c", subcore_axis_name="s")

    @functools.partial(
        pl.kernel,
        mesh=mesh,
        out_type=jax.ShapeDtypeStruct((total, 2 * embed), jnp.float32),
        scratch_types=[
            pltpu.VMEM((n_chunks, _CHUNK), jnp.int32),
            pltpu.VMEM((n_chunks, _CHUNK), jnp.int32),
            pltpu.VMEM((b_per_w, 2 * embed), jnp.float32),
            pltpu.SemaphoreType.DMA,
            pltpu.SemaphoreType.DMA,
        ],
    )
    def gather_sc(user_hbm, item_hbm, pidx_u_hbm, pidx_i_hbm, out_hbm,
                  idx_u_v, idx_i_v, buf, gsem, wsem):
        wid = lax.axis_index("s") * num_cores + lax.axis_index("c")
        base = wid * b_per_w

        pltpu.sync_copy(pidx_u_hbm.at[wid], idx_u_v)
        pltpu.sync_copy(pidx_i_hbm.at[wid], idx_i_v)

        copies = [
            pltpu.async_copy(
                user_hbm.at[idx_u_v.at[c]],
                buf.at[pl.ds(c * _CHUNK, _CHUNK)],
                gsem,
            )
            for c in range(n_chunks)
        ]
        for cp in copies:
            cp.wait()
        w = pltpu.async_copy(buf, out_hbm.at[pl.ds(base, b_per_w)], wsem)
        w.wait()

        copies = [
            pltpu.async_copy(
                item_hbm.at[idx_i_v.at[c]],
                buf.at[pl.ds(c * _CHUNK, _CHUNK)],
                gsem,
            )
            for c in range(n_chunks)
        ]
        for cp in copies:
            cp.wait()
        w = pltpu.async_copy(
            buf, out_hbm.at[pl.ds(batch + base, b_per_w)], wsem)
        w.wait()

    blk = 2048
    n_blk = total // blk

    def select_tc(rows_ref, bits_ref, o_ref):
        r = rows_ref[...]
        b = bits_ref[...] > 0
        o_ref[...] = jnp.where(b, r[:, embed:], r[:, :embed])

    select = pl.pallas_call(
        select_tc,
        grid=(n_blk,),
        in_specs=[
            pl.BlockSpec((blk, 2 * embed), lambda i: (i, 0)),
            pl.BlockSpec((blk, 1), lambda i: (i, 0)),
        ],
        out_specs=pl.BlockSpec((blk, embed), lambda i: (i, 0)),
        out_shape=jax.ShapeDtypeStruct((total, embed), jnp.float32),
    )

    def call(embed_user, embed_item, idx_user, idx_item):
        u2 = embed_user.reshape(n_user // 2, 2 * embed)
        i2 = embed_item.reshape(n_item // 2, 2 * embed)
        idx_u = idx_user.astype(jnp.int32)
        idx_i = idx_item.astype(jnp.int32)
        pidx_u = (idx_u >> 1).reshape(num_workers, n_chunks, _CHUNK)
        pidx_i = (idx_i >> 1).reshape(num_workers, n_chunks, _CHUNK)
        bits = jnp.concatenate([idx_u & 1, idx_i & 1]).reshape(total, 1)
        rows = gather_sc(u2, i2, pidx_u, pidx_i)
        return select(rows, bits)

    return call


def kernel(embed_user, embed_item, idx_user, idx_item):
    n_user, embed = embed_user.shape
    n_item = embed_item.shape[0]
    batch = idx_user.shape[0]
    return _build(n_user, n_item, batch, embed)(
        embed_user, embed_item, idx_user, idx_item)
